# Initial kernel scaffold; baseline (speedup 1.0000x reference)
import jax
import jax.numpy as jnp
from jax.experimental import pallas as pl


def kernel(x, table):
    raise NotImplementedError("write your pallas kernel here")


# speedup vs baseline: 2.9728x; 2.9728x over previous
import functools

import jax
import jax.numpy as jnp
from jax import lax
from jax.experimental import pallas as pl
from jax.experimental.pallas import tpu as pltpu
from jax.experimental.pallas import tpu_sc as plsc

VOCAB = 100000
D = 128

_info = plsc.get_sparse_core_info()
_NC, _NS = _info.num_cores, _info.num_subcores
_NW = _NC * _NS

_B = 4096 * 50
_BPW = _B // _NW
_C = 128
_NCHUNK = _BPW // _C


def _make_kernel():
    mesh = plsc.VectorSubcoreMesh(core_axis_name="c", subcore_axis_name="s")

    @functools.partial(
        pl.kernel,
        out_type=jax.ShapeDtypeStruct((_B, D), jnp.float32),
        mesh=mesh,
        scratch_types=[
            pltpu.VMEM((_NCHUNK, _C), jnp.int32),
            pltpu.VMEM((_C, D), jnp.float32),
            pltpu.SemaphoreType.DMA,
        ],
    )
    def emb(idx_hbm, table_hbm, out_hbm, idx_v, rows_v, sem):
        wid = lax.axis_index("s") * _NC + lax.axis_index("c")
        base = wid * _BPW
        pltpu.sync_copy(idx_hbm.at[wid], idx_v)

        def chunk(i, carry):
            pltpu.async_copy(table_hbm.at[idx_v.at[i]], rows_v, sem).wait()
            pltpu.sync_copy(rows_v, out_hbm.at[pl.ds(base + i * _C, _C)])
            return carry

        lax.fori_loop(0, _NCHUNK, chunk, 0)

    return emb


_emb = _make_kernel()


def kernel(x, table):
    idx = x.reshape(_NW, _NCHUNK, _C).astype(jnp.int32)
    out = _emb(idx, table)
    return out.reshape(4096, 50, D)


# speedup vs baseline: 3.3497x; 1.1268x over previous
import functools

import jax
import jax.numpy as jnp
from jax import lax
from jax.experimental import pallas as pl
from jax.experimental.pallas import tpu as pltpu
from jax.experimental.pallas import tpu_sc as plsc

VOCAB = 100000
D = 128

_info = plsc.get_sparse_core_info()
_NC, _NS = _info.num_cores, _info.num_subcores
_NW = _NC * _NS

_B = 4096 * 50
_BPW = _B // _NW
_C = 128
_NCHUNK = _BPW // _C


def _make_kernel():
    mesh = plsc.VectorSubcoreMesh(core_axis_name="c", subcore_axis_name="s")

    @functools.partial(
        pl.kernel,
        out_type=jax.ShapeDtypeStruct((_B, D), jnp.float32),
        mesh=mesh,
        scratch_types=[
            pltpu.VMEM((_NCHUNK, _C), jnp.int32),
            pltpu.VMEM((2, _C, D), jnp.float32),
            pltpu.SemaphoreType.DMA,
        ],
    )
    def emb(idx_hbm, table_hbm, out_hbm, idx_v, rows_v, sem):
        wid = lax.axis_index("s") * _NC + lax.axis_index("c")
        base = wid * _BPW
        pltpu.sync_copy(idx_hbm.at[wid], idx_v)

        pltpu.async_copy(table_hbm.at[idx_v.at[0]], rows_v.at[0], sem)

        def outer(it, carry):
            g = it * 2
            for b in range(2):
                i = g + b

                @pl.when(i + 1 < _NCHUNK)
                def _():
                    pltpu.async_copy(
                        table_hbm.at[idx_v.at[i + 1]], rows_v.at[1 - b], sem
                    )

                pltpu.make_async_copy(
                    out_hbm.at[pl.ds(0, _C)], rows_v.at[b], sem
                ).wait()
                pltpu.sync_copy(rows_v.at[b], out_hbm.at[pl.ds(base + i * _C, _C)])
            return carry

        lax.fori_loop(0, _NCHUNK // 2, outer, 0)

    return emb


_emb = _make_kernel()


def kernel(x, table):
    idx = x.reshape(_NW, _NCHUNK, _C).astype(jnp.int32)
    out = _emb(idx, table)
    return out.reshape(4096, 50, D)


# speedup vs baseline: 5.1312x; 1.5318x over previous
import functools

import jax
import jax.numpy as jnp
from jax import lax
from jax.experimental import pallas as pl
from jax.experimental.pallas import tpu as pltpu
from jax.experimental.pallas import tpu_sc as plsc

VOCAB = 100000
D = 128
SEQ = 4096
W = 50
WPAD = 128

_info = plsc.get_sparse_core_info()
_NC, _NS = _info.num_cores, _info.num_subcores
_NW = _NC * _NS
_SPW = SEQ // _NW


def _make_kernel():
    mesh = plsc.VectorSubcoreMesh(core_axis_name="c", subcore_axis_name="s")

    @functools.partial(
        pl.kernel,
        out_type=jax.ShapeDtypeStruct((SEQ, W, D), jnp.float32),
        mesh=mesh,
        scratch_types=[
            pltpu.VMEM((_SPW, WPAD), jnp.int32),
            pltpu.VMEM((2, W, D), jnp.float32),
            pltpu.SemaphoreType.DMA,
        ],
        compiler_params=pltpu.CompilerParams(use_tc_tiling_on_sc=True),
    )
    def emb(idx_hbm, table_hbm, out_hbm, idx_v, rows_v, sem):
        wid = lax.axis_index("s") * _NC + lax.axis_index("c")
        base = wid * _SPW
        pltpu.sync_copy(idx_hbm.at[pl.ds(base, _SPW)], idx_v)

        pltpu.async_copy(
            table_hbm.at[idx_v.at[0, pl.ds(0, W)]], rows_v.at[0], sem
        )

        def outer(it, carry):
            g = it * 2
            for b in range(2):
                i = g + b

                @pl.when(i + 1 < _SPW)
                def _():
                    pltpu.async_copy(
                        table_hbm.at[idx_v.at[i + 1, pl.ds(0, W)]],
                        rows_v.at[1 - b],
                        sem,
                    )

                pltpu.make_async_copy(
                    out_hbm.at[0], rows_v.at[b], sem
                ).wait()
                pltpu.sync_copy(rows_v.at[b], out_hbm.at[base + i])
            return carry

        lax.fori_loop(0, _SPW // 2, outer, 0)

    return emb


_emb = _make_kernel()


def kernel(x, table):
    idx = jnp.pad(x.astype(jnp.int32), ((0, 0), (0, WPAD - W)))
    return _emb(idx, table)


# speedup vs baseline: 10.4038x; 2.0275x over previous
import functools

import jax
import jax.numpy as jnp
from jax import lax
from jax.experimental import pallas as pl
from jax.experimental.pallas import tpu as pltpu
from jax.experimental.pallas import tpu_sc as plsc

VOCAB = 100000
D = 128
SEQ = 4096
W = 50

_info = plsc.get_sparse_core_info()
_NC, _NS = _info.num_cores, _info.num_subcores
_NW = _NC * _NS
_B = SEQ * W
_BPW = _B // _NW
_C = 128
_NCHUNK = _BPW // _C


def _make_kernel():
    mesh = plsc.VectorSubcoreMesh(core_axis_name="c", subcore_axis_name="s")

    @functools.partial(
        pl.kernel,
        out_type=jax.ShapeDtypeStruct((_B, D), jnp.float32),
        mesh=mesh,
        scratch_types=[
            pltpu.VMEM((_NCHUNK, _C), jnp.int32),
            pltpu.VMEM((2, _C, D), jnp.float32),
            pltpu.SemaphoreType.DMA,
        ],
        compiler_params=pltpu.CompilerParams(use_tc_tiling_on_sc=True),
    )
    def emb(idx_hbm, table_hbm, out_hbm, idx_v, rows_v, sem):
        wid = lax.axis_index("s") * _NC + lax.axis_index("c")
        base = wid * _BPW
        pltpu.sync_copy(idx_hbm.at[wid], idx_v)

        pltpu.async_copy(table_hbm.at[idx_v.at[0]], rows_v.at[0], sem)

        def outer(it, carry):
            g = it * 2
            for b in range(2):
                i = g + b

                @pl.when(i + 1 < _NCHUNK)
                def _():
                    pltpu.async_copy(
                        table_hbm.at[idx_v.at[i + 1]], rows_v.at[1 - b], sem
                    )

                pltpu.make_async_copy(
                    out_hbm.at[pl.ds(0, _C)], rows_v.at[b], sem
                ).wait()
                pltpu.sync_copy(rows_v.at[b], out_hbm.at[pl.ds(base + i * _C, _C)])
            return carry

        lax.fori_loop(0, _NCHUNK // 2, outer, 0)

    return emb


_emb = _make_kernel()


def kernel(x, table):
    idx = x.T.astype(jnp.int32).reshape(_NW, _NCHUNK, _C)
    out = _emb(idx, table)
    return out.reshape(W, SEQ, D).transpose(1, 0, 2)


# speedup vs baseline: 10.5606x; 1.0151x over previous
import functools

import jax
import jax.numpy as jnp
from jax import lax
from jax.experimental import pallas as pl
from jax.experimental.pallas import tpu as pltpu
from jax.experimental.pallas import tpu_sc as plsc

VOCAB = 100000
D = 128
SEQ = 4096
W = 50

_info = plsc.get_sparse_core_info()
_NC, _NS = _info.num_cores, _info.num_subcores
_NW = _NC * _NS
_B = SEQ * W
_BPW = _B // _NW
_C = 128
_NCHUNK = _BPW // _C
_NBUF = 5


def _make_kernel():
    mesh = plsc.VectorSubcoreMesh(core_axis_name="c", subcore_axis_name="s")

    @functools.partial(
        pl.kernel,
        out_type=jax.ShapeDtypeStruct((_B, D), jnp.float32),
        mesh=mesh,
        scratch_types=[
            pltpu.VMEM((_NCHUNK, _C), jnp.int32),
            pltpu.VMEM((_NBUF, _C, D), jnp.float32),
            pltpu.SemaphoreType.DMA,
            pltpu.SemaphoreType.DMA,
        ],
        compiler_params=pltpu.CompilerParams(use_tc_tiling_on_sc=True),
    )
    def emb(idx_hbm, table_hbm, out_hbm, idx_v, rows_v, gsem, wsem):
        wid = lax.axis_index("s") * _NC + lax.axis_index("c")
        base = wid * _BPW
        pltpu.sync_copy(idx_hbm.at[wid], idx_v)

        for j in range(3):
            pltpu.async_copy(table_hbm.at[idx_v.at[j]], rows_v.at[j], gsem)

        def outer(it, carry):
            g = it * _NBUF
            for b in range(_NBUF):
                i = g + b

                @pl.when(i >= 2)
                def _():
                    pltpu.make_async_copy(
                        out_hbm.at[pl.ds(0, _C)], rows_v.at[b], wsem
                    ).wait()

                @pl.when(i + 3 < _NCHUNK)
                def _():
                    pltpu.async_copy(
                        table_hbm.at[idx_v.at[i + 3]],
                        rows_v.at[(b + 3) % _NBUF],
                        gsem,
                    )

                pltpu.make_async_copy(
                    out_hbm.at[pl.ds(0, _C)], rows_v.at[b], gsem
                ).wait()
                pltpu.async_copy(
                    rows_v.at[b], out_hbm.at[pl.ds(base + i * _C, _C)], wsem
                )
            return carry

        lax.fori_loop(0, _NCHUNK // _NBUF, outer, 0)
        for j in range(2):
            pltpu.make_async_copy(
                out_hbm.at[pl.ds(0, _C)], rows_v.at[j], wsem
            ).wait()

    return emb


_emb = _make_kernel()


def kernel(x, table):
    idx = x.T.astype(jnp.int32).reshape(_NW, _NCHUNK, _C)
    out = _emb(idx, table)
    return out.reshape(W, SEQ, D).transpose(1, 0, 2)


# speedup vs baseline: 10.5626x; 1.0002x over previous
import functools

import jax
import jax.numpy as jnp
from jax import lax
from jax.experimental import pallas as pl
from jax.experimental.pallas import tpu as pltpu
from jax.experimental.pallas import tpu_sc as plsc

VOCAB = 100000
D = 128
SEQ = 4096
W = 50

_info = plsc.get_sparse_core_info()
_NC, _NS = _info.num_cores, _info.num_subcores
_NW = _NC * _NS
_B = SEQ * W
_BPW = _B // _NW
_C = 128
_NCHUNK = _BPW // _C
_NBUF = 3
_NSTEP = _NCHUNK // 2


def _make_kernel():
    mesh = plsc.VectorSubcoreMesh(core_axis_name="c", subcore_axis_name="s")

    @functools.partial(
        pl.kernel,
        out_type=jax.ShapeDtypeStruct((_B, D), jnp.float32),
        mesh=mesh,
        scratch_types=[
            pltpu.VMEM((_NCHUNK, _C), jnp.int32),
            pltpu.VMEM((_NBUF, 2 * _C, D), jnp.float32),
            pltpu.SemaphoreType.DMA,
            pltpu.SemaphoreType.DMA,
        ],
        compiler_params=pltpu.CompilerParams(use_tc_tiling_on_sc=True),
    )
    def emb(idx_hbm, table_hbm, out_hbm, idx_v, rows_v, gsem, wsem):
        wid = lax.axis_index("s") * _NC + lax.axis_index("c")
        base = wid * _BPW
        pltpu.sync_copy(idx_hbm.at[wid], idx_v)

        def gathers(j, b):
            pltpu.async_copy(
                table_hbm.at[idx_v.at[2 * j]], rows_v.at[b, pl.ds(0, _C)], gsem
            )
            pltpu.async_copy(
                table_hbm.at[idx_v.at[2 * j + 1]],
                rows_v.at[b, pl.ds(_C, _C)],
                gsem,
            )

        for j in range(2):
            gathers(j, j)

        def outer(it, carry):
            g = it * _NBUF
            for b in range(_NBUF):
                j = g + b

                @pl.when(j >= 1)
                def _():
                    pltpu.make_async_copy(
                        out_hbm.at[pl.ds(0, 2 * _C)], rows_v.at[b], wsem
                    ).wait()

                @pl.when(j + 2 < _NSTEP)
                def _():
                    gathers(j + 2, (b + 2) % _NBUF)

                pltpu.make_async_copy(
                    out_hbm.at[pl.ds(0, 2 * _C)], rows_v.at[b], gsem
                ).wait()
                pltpu.async_copy(
                    rows_v.at[b],
                    out_hbm.at[pl.ds(base + j * 2 * _C, 2 * _C)],
                    wsem,
                )
            return carry

        lax.fori_loop(0, _NSTEP // _NBUF, outer, 0)

        jlast = _NSTEP - 1
        blast = jlast % _NBUF
        pltpu.make_async_copy(
            out_hbm.at[pl.ds(0, 2 * _C)], rows_v.at[blast], wsem
        ).wait()
        pltpu.make_async_copy(
            out_hbm.at[pl.ds(0, 2 * _C)], rows_v.at[blast], gsem
        ).wait()
        pltpu.async_copy(
            rows_v.at[blast],
            out_hbm.at[pl.ds(base + jlast * 2 * _C, 2 * _C)],
            wsem,
        )
        pltpu.make_async_copy(
            out_hbm.at[pl.ds(0, 2 * _C)], rows_v.at[blast], wsem
        ).wait()

    return emb


_emb = _make_kernel()


def kernel(x, table):
    idx = x.T.astype(jnp.int32).reshape(_NW, _NCHUNK, _C)
    out = _emb(idx, table)
    return out.reshape(W, SEQ, D).transpose(1, 0, 2)


# speedup vs baseline: 10.8300x; 1.0253x over previous
import functools

import jax
import jax.numpy as jnp
from jax import lax
from jax.experimental import pallas as pl
from jax.experimental.pallas import tpu as pltpu
from jax.experimental.pallas import tpu_sc as plsc

VOCAB = 100000
D = 128
SEQ = 4096
W = 50

_info = plsc.get_sparse_core_info()
_NC, _NS = _info.num_cores, _info.num_subcores
_NW = _NC * _NS
_B = SEQ * W
_BPW = _B // _NW
_C = 128
_NCHUNK = _BPW // _C
_NBUF = 5


def _make_kernel():
    mesh = plsc.VectorSubcoreMesh(core_axis_name="c", subcore_axis_name="s")

    @functools.partial(
        pl.kernel,
        out_type=jax.ShapeDtypeStruct((_B, D), jnp.float32),
        mesh=mesh,
        scratch_types=[
            pltpu.VMEM((_NCHUNK, _C), jnp.int32),
            pltpu.VMEM((_NBUF, _C, D), jnp.float32),
            pltpu.SemaphoreType.DMA,
            pltpu.SemaphoreType.DMA,
        ],
        compiler_params=pltpu.CompilerParams(use_tc_tiling_on_sc=True),
    )
    def emb(idx_hbm, table_hbm, out_hbm, idx_v, rows_v, gsem, wsem):
        wid = lax.axis_index("s") * _NC + lax.axis_index("c")
        col = wid * _C
        pltpu.sync_copy(idx_hbm.at[pl.ds(0, W), pl.ds(col, _C)], idx_v)

        for j in range(3):
            pltpu.async_copy(table_hbm.at[idx_v.at[j]], rows_v.at[j], gsem)

        def outer(it, carry):
            g = it * _NBUF
            for b in range(_NBUF):
                i = g + b

                @pl.when(i >= 2)
                def _():
                    pltpu.make_async_copy(
                        out_hbm.at[pl.ds(0, _C)], rows_v.at[b], wsem
                    ).wait()

                @pl.when(i + 3 < _NCHUNK)
                def _():
                    pltpu.async_copy(
                        table_hbm.at[idx_v.at[i + 3]],
                        rows_v.at[(b + 3) % _NBUF],
                        gsem,
                    )

                pltpu.make_async_copy(
                    out_hbm.at[pl.ds(0, _C)], rows_v.at[b], gsem
                ).wait()
                pltpu.async_copy(
                    rows_v.at[b], out_hbm.at[pl.ds(i * SEQ + col, _C)], wsem
                )
            return carry

        lax.fori_loop(0, _NCHUNK // _NBUF, outer, 0)
        for j in range(2):
            pltpu.make_async_copy(
                out_hbm.at[pl.ds(0, _C)], rows_v.at[j], wsem
            ).wait()

    return emb


_emb = _make_kernel()


def kernel(x, table):
    idx = x.T.astype(jnp.int32)
    out = _emb(idx, table)
    return out.reshape(W, SEQ, D).transpose(1, 0, 2)


# speedup vs baseline: 10.8384x; 1.0008x over previous
import functools

import jax
import jax.numpy as jnp
from jax import lax
from jax.experimental import pallas as pl
from jax.experimental.pallas import tpu as pltpu
from jax.experimental.pallas import tpu_sc as plsc

VOCAB = 100000
D = 128
SEQ = 4096
W = 50

_info = plsc.get_sparse_core_info()
_NC, _NS = _info.num_cores, _info.num_subcores
_NW = _NC * _NS
_B = SEQ * W
_BPW = _B // _NW
_C = 128
_NCHUNK = _BPW // _C
_NBUF = 5


def _make_kernel():
    mesh = plsc.VectorSubcoreMesh(core_axis_name="c", subcore_axis_name="s")

    @functools.partial(
        pl.kernel,
        out_type=jax.ShapeDtypeStruct((_B, D), jnp.float32),
        mesh=mesh,
        scratch_types=[
            pltpu.VMEM((_NCHUNK, _C), jnp.int32),
            pltpu.VMEM((_NBUF, _C, D), jnp.float32),
            pltpu.SemaphoreType.DMA,
            pltpu.SemaphoreType.DMA,
        ],
        compiler_params=pltpu.CompilerParams(use_tc_tiling_on_sc=True),
    )
    def emb(idx_hbm, table_hbm, out_hbm, idx_v, rows_v, gsem, wsem):
        wid = lax.axis_index("s") * _NC + lax.axis_index("c")
        col = wid * _C
        pltpu.sync_copy(idx_hbm.at[pl.ds(0, W), pl.ds(col, _C)], idx_v)

        for j in range(3):
            pltpu.async_copy(table_hbm.at[idx_v.at[j]], rows_v.at[j], gsem)

        def body(i, carry):
            b = lax.rem(i, _NBUF)

            @pl.when(i >= 2)
            def _():
                pltpu.make_async_copy(
                    out_hbm.at[pl.ds(0, _C)], rows_v.at[b], wsem
                ).wait()

            @pl.when(i + 3 < _NCHUNK)
            def _():
                pltpu.async_copy(
                    table_hbm.at[idx_v.at[i + 3]],
                    rows_v.at[lax.rem(i + 3, _NBUF)],
                    gsem,
                )

            pltpu.make_async_copy(
                out_hbm.at[pl.ds(0, _C)], rows_v.at[b], gsem
            ).wait()
            pltpu.async_copy(
                rows_v.at[b], out_hbm.at[pl.ds(i * SEQ + col, _C)], wsem
            )
            return carry

        lax.fori_loop(0, _NCHUNK, body, 0)
        for j in range(2):
            pltpu.make_async_copy(
                out_hbm.at[pl.ds(0, _C)], rows_v.at[j], wsem
            ).wait()

    return emb


_emb = _make_kernel()


def kernel(x, table):
    idx = x.T.astype(jnp.int32)
    out = _emb(idx, table)
    return out.reshape(W, SEQ, D).transpose(1, 0, 2)


# speedup vs baseline: 10.8668x; 1.0026x over previous
import functools

import jax
import jax.numpy as jnp
from jax import lax
from jax.experimental import pallas as pl
from jax.experimental.pallas import tpu as pltpu
from jax.experimental.pallas import tpu_sc as plsc

VOCAB = 100000
D = 128
SEQ = 4096
W = 50

_info = plsc.get_sparse_core_info()
_NC, _NS = _info.num_cores, _info.num_subcores
_NW = _NC * _NS
_B = SEQ * W
_BPW = _B // _NW
_C = 128
_NCHUNK = _BPW // _C
_NBUF = 5


def _make_kernel():
    mesh = plsc.VectorSubcoreMesh(core_axis_name="c", subcore_axis_name="s")

    @functools.partial(
        pl.kernel,
        out_type=jax.ShapeDtypeStruct((_B, D), jnp.float32),
        mesh=mesh,
        scratch_types=[
            pltpu.VMEM((_NCHUNK, _C), jnp.int32),
            pltpu.VMEM((_NBUF, _C, D), jnp.float32),
            pltpu.SemaphoreType.DMA,
            pltpu.SemaphoreType.DMA,
        ],
        compiler_params=pltpu.CompilerParams(
            use_tc_tiling_on_sc=True,
            disable_bounds_checks=True,
            skip_device_barrier=True,
        ),
    )
    def emb(idx_hbm, table_hbm, out_hbm, idx_v, rows_v, gsem, wsem):
        wid = lax.axis_index("s") * _NC + lax.axis_index("c")
        col = wid * _C
        pltpu.sync_copy(idx_hbm.at[pl.ds(0, W), pl.ds(col, _C)], idx_v)

        for j in range(3):
            pltpu.async_copy(table_hbm.at[idx_v.at[j]], rows_v.at[j], gsem)

        def body(i, carry):
            b = lax.rem(i, _NBUF)

            @pl.when(i >= 2)
            def _():
                pltpu.make_async_copy(
                    out_hbm.at[pl.ds(0, _C)], rows_v.at[b], wsem
                ).wait()

            @pl.when(i + 3 < _NCHUNK)
            def _():
                pltpu.async_copy(
                    table_hbm.at[idx_v.at[i + 3]],
                    rows_v.at[lax.rem(i + 3, _NBUF)],
                    gsem,
                )

            pltpu.make_async_copy(
                out_hbm.at[pl.ds(0, _C)], rows_v.at[b], gsem
            ).wait()
            pltpu.async_copy(
                rows_v.at[b], out_hbm.at[pl.ds(i * SEQ + col, _C)], wsem
            )
            return carry

        lax.fori_loop(0, _NCHUNK, body, 0)
        for j in range(2):
            pltpu.make_async_copy(
                out_hbm.at[pl.ds(0, _C)], rows_v.at[j], wsem
            ).wait()

    return emb


_emb = _make_kernel()


def kernel(x, table):
    idx = x.T.astype(jnp.int32)
    out = _emb(idx, table)
    return out.reshape(W, SEQ, D).transpose(1, 0, 2)
